# baseline (speedup 1.0000x reference)
import functools

import jax
import jax.numpy as jnp
from jax import lax
from jax.experimental import pallas as pl
from jax.experimental.pallas import tpu as pltpu
from jax.experimental.pallas import tpu_sc as plsc

MAX_POS = 8192
N_EMBD = 1024
BATCH = 4
SEQ = 8192

_INFO = plsc.get_sparse_core_info()
_NC = _INFO.num_cores
_NS = _INFO.num_subcores
_NW = _NC * _NS
_B = BATCH * SEQ
_B_PER_W = _B // _NW
_C = 32
_NBUF = 2
_N_CHUNKS = _B_PER_W // _C


def _sc_gather(idx_flat, table):
    mesh = plsc.VectorSubcoreMesh(core_axis_name="c", subcore_axis_name="s")

    @functools.partial(
        pl.kernel,
        mesh=mesh,
        out_type=jax.ShapeDtypeStruct((_B, N_EMBD), jnp.float32),
        scratch_types=[
            pltpu.VMEM((_B_PER_W,), jnp.int32),
            pltpu.VMEM_SHARED((_NS, _NBUF, _C, N_EMBD), jnp.float32),
            pltpu.SemaphoreType.DMA((_NBUF,)),
            pltpu.SemaphoreType.DMA((_NBUF,)),
        ],
    )
    def k(idx_hbm, table_hbm, out_hbm, idx_v, rows_s, gsem, wsem):
        sid = lax.axis_index("s")
        wid = sid * _NC + lax.axis_index("c")
        base = wid * _B_PER_W
        pltpu.sync_copy(idx_hbm.at[pl.ds(base, _B_PER_W)], idx_v)

        def start_gather(i, b):
            pltpu.async_copy(
                table_hbm.at[idx_v.at[pl.ds(i * _C, _C)]],
                rows_s.at[sid, b],
                gsem.at[b],
            )

        def wait_gather(i, b):
            pltpu.make_async_copy(
                table_hbm.at[idx_v.at[pl.ds(i * _C, _C)]],
                rows_s.at[sid, b],
                gsem.at[b],
            ).wait()

        def start_write(i, b):
            pltpu.async_copy(
                rows_s.at[sid, b],
                out_hbm.at[pl.ds(base + i * _C, _C)],
                wsem.at[b],
            )

        def wait_write(i, b):
            pltpu.make_async_copy(
                rows_s.at[sid, b],
                out_hbm.at[pl.ds(base + i * _C, _C)],
                wsem.at[b],
            ).wait()

        for b in range(_NBUF):
            start_gather(b, b)

        def body(g, carry):
            for b in range(_NBUF):
                i = g * _NBUF + b
                wait_gather(i, b)
                start_write(i, b)
                wait_write(i, b)
                start_gather(i + _NBUF, b)
            return carry

        lax.fori_loop(0, (_N_CHUNKS - _NBUF) // _NBUF, body, 0)

        tail = _N_CHUNKS - _NBUF
        for b in range(_NBUF):
            wait_gather(tail + b, b)
            start_write(tail + b, b)
        for b in range(_NBUF):
            wait_write(tail + b, b)

    return k(idx_flat, table)


_TC_R = 256


def _tc_gather(idx, table, m):
    def body(idx_ref, table_ref, out_ref):
        i = pl.program_id(0)

        def row(r, carry):
            out_ref[r, :] = table_ref[idx_ref[i * _TC_R + r], :]
            return carry

        lax.fori_loop(0, _TC_R, row, 0)

    return pl.pallas_call(
        body,
        grid_spec=pltpu.PrefetchScalarGridSpec(
            num_scalar_prefetch=1,
            grid=(m // _TC_R,),
            in_specs=[
                pl.BlockSpec((MAX_POS, N_EMBD), lambda i, idx_ref: (0, 0)),
            ],
            out_specs=pl.BlockSpec((_TC_R, N_EMBD), lambda i, idx_ref: (i, 0)),
        ),
        out_shape=jax.ShapeDtypeStruct((m, N_EMBD), jnp.float32),
    )(idx, table)


def kernel(position_ids, table):
    idx_flat = position_ids.reshape(_B).astype(jnp.int32)
    out = _tc_gather(idx_flat, table, _B)
    return out.reshape(BATCH, SEQ, N_EMBD)


# speedup vs baseline: 1.5387x; 1.5387x over previous
import functools

import jax
import jax.numpy as jnp
from jax import lax
from jax.experimental import pallas as pl
from jax.experimental.pallas import tpu as pltpu
from jax.experimental.pallas import tpu_sc as plsc

MAX_POS = 8192
N_EMBD = 1024
BATCH = 4
SEQ = 8192

_INFO = plsc.get_sparse_core_info()
_NC = _INFO.num_cores
_NS = _INFO.num_subcores
_NW = _NC * _NS
_B = BATCH * SEQ
_C = 16
_NBUF = 4


def _sc_gather(idx_flat, table, n_rows):
    per_w = n_rows // _NW
    n_chunks = per_w // _C
    mesh = plsc.VectorSubcoreMesh(core_axis_name="c", subcore_axis_name="s")

    @functools.partial(
        pl.kernel,
        mesh=mesh,
        out_type=jax.ShapeDtypeStruct((n_rows, N_EMBD), jnp.float32),
        scratch_types=[
            pltpu.VMEM((per_w,), jnp.int32),
            pltpu.VMEM((_NBUF, _C, N_EMBD), jnp.float32),
            pltpu.SemaphoreType.DMA((_NBUF,)),
            pltpu.SemaphoreType.DMA((_NBUF,)),
        ],
    )
    def k(idx_hbm, table_hbm, out_hbm, idx_v, rows_v, gsem, wsem):
        wid = lax.axis_index("s") * _NC + lax.axis_index("c")
        base = wid * per_w
        pltpu.sync_copy(idx_hbm.at[pl.ds(base, per_w)], idx_v)

        def start_gather(i, b):
            pltpu.async_copy(
                table_hbm.at[idx_v.at[pl.ds(i * _C, _C)]],
                rows_v.at[b],
                gsem.at[b],
            )

        def wait_gather(i, b):
            pltpu.make_async_copy(
                table_hbm.at[idx_v.at[pl.ds(i * _C, _C)]],
                rows_v.at[b],
                gsem.at[b],
            ).wait()

        def start_write(i, b):
            pltpu.async_copy(
                rows_v.at[b],
                out_hbm.at[pl.ds(base + i * _C, _C)],
                wsem.at[b],
            )

        def wait_write(i, b):
            pltpu.make_async_copy(
                rows_v.at[b],
                out_hbm.at[pl.ds(base + i * _C, _C)],
                wsem.at[b],
            ).wait()

        for b in range(_NBUF):
            start_gather(b, b)

        def body(g, carry):
            for b in range(_NBUF):
                i = g * _NBUF + b
                wait_gather(i, b)
                start_write(i, b)
                wait_write(i, b)
                start_gather(i + _NBUF, b)
            return carry

        lax.fori_loop(0, (n_chunks - _NBUF) // _NBUF, body, 0)

        tail = n_chunks - _NBUF
        for b in range(_NBUF):
            wait_gather(tail + b, b)
            start_write(tail + b, b)
        for b in range(_NBUF):
            wait_write(tail + b, b)

    return k(idx_flat, table)


_TC_R = 256


def _tc_gather(idx, table, m):
    def body(idx_ref, table_ref, out_ref):
        i = pl.program_id(0)

        def row(r, carry):
            out_ref[r, :] = table_ref[idx_ref[i * _TC_R + r], :]
            return carry

        lax.fori_loop(0, _TC_R, row, 0)

    return pl.pallas_call(
        body,
        grid_spec=pltpu.PrefetchScalarGridSpec(
            num_scalar_prefetch=1,
            grid=(m // _TC_R,),
            in_specs=[
                pl.BlockSpec((MAX_POS, N_EMBD), lambda i, idx_ref: (0, 0)),
            ],
            out_specs=pl.BlockSpec((_TC_R, N_EMBD), lambda i, idx_ref: (i, 0)),
        ),
        out_shape=jax.ShapeDtypeStruct((m, N_EMBD), jnp.float32),
    )(idx, table)


_TC_SHARE = 8192


def kernel(position_ids, table):
    idx_flat = position_ids.reshape(_B).astype(jnp.int32)
    sc_out = _sc_gather(idx_flat[_TC_SHARE:], table, _B - _TC_SHARE)
    tc_out = _tc_gather(idx_flat[:_TC_SHARE], table, _TC_SHARE)
    out = jnp.concatenate([tc_out, sc_out], axis=0)
    return out.reshape(BATCH, SEQ, N_EMBD)


# speedup vs baseline: 2.9237x; 1.9001x over previous
import functools

import jax
import jax.numpy as jnp
from jax import lax
from jax.experimental import pallas as pl
from jax.experimental.pallas import tpu as pltpu
from jax.experimental.pallas import tpu_sc as plsc

MAX_POS = 8192
N_EMBD = 1024
BATCH = 4
SEQ = 8192

_INFO = plsc.get_sparse_core_info()
_NC = _INFO.num_cores
_NS = _INFO.num_subcores
_NW = _NC * _NS
_B = BATCH * SEQ
_C = 16
_NBUF = 4


def _sc_gather(idx_flat, table, n_rows):
    per_w = n_rows // _NW
    n_chunks = per_w // _C
    mesh = plsc.VectorSubcoreMesh(core_axis_name="c", subcore_axis_name="s")

    @functools.partial(
        pl.kernel,
        mesh=mesh,
        out_type=jax.ShapeDtypeStruct((n_rows, N_EMBD), jnp.float32),
        scratch_types=[
            pltpu.VMEM((per_w,), jnp.int32),
            pltpu.VMEM((_NBUF, _C, N_EMBD), jnp.float32),
            pltpu.SemaphoreType.DMA((_NBUF,)),
            pltpu.SemaphoreType.DMA((_NBUF,)),
        ],
    )
    def k(idx_hbm, table_hbm, out_hbm, idx_v, rows_v, gsem, wsem):
        wid = lax.axis_index("s") * _NC + lax.axis_index("c")
        base = wid * per_w
        pltpu.sync_copy(idx_hbm.at[pl.ds(base, per_w)], idx_v)

        def start_gather(i, b):
            pltpu.async_copy(
                table_hbm.at[idx_v.at[pl.ds(i * _C, _C)]],
                rows_v.at[b],
                gsem.at[b],
            )

        def wait_gather(i, b):
            pltpu.make_async_copy(
                table_hbm.at[idx_v.at[pl.ds(i * _C, _C)]],
                rows_v.at[b],
                gsem.at[b],
            ).wait()

        def start_write(i, b):
            pltpu.async_copy(
                rows_v.at[b],
                out_hbm.at[pl.ds(base + i * _C, _C)],
                wsem.at[b],
            )

        def wait_write(i, b):
            pltpu.make_async_copy(
                rows_v.at[b],
                out_hbm.at[pl.ds(base + i * _C, _C)],
                wsem.at[b],
            ).wait()

        for b in range(_NBUF):
            start_gather(b, b)

        def body(g, carry):
            for b in range(_NBUF):
                i = g * _NBUF + b
                wait_gather(i, b)
                start_write(i, b)
                wait_write(i, b)
                start_gather(i + _NBUF, b)
            return carry

        lax.fori_loop(0, (n_chunks - _NBUF) // _NBUF, body, 0)

        tail = n_chunks - _NBUF
        for b in range(_NBUF):
            wait_gather(tail + b, b)
            start_write(tail + b, b)
        for b in range(_NBUF):
            wait_write(tail + b, b)

    return k(idx_flat, table)


def kernel(position_ids, table):
    idx_flat = position_ids.reshape(_B).astype(jnp.int32)
    out = _sc_gather(idx_flat, table, _B)
    return out.reshape(BATCH, SEQ, N_EMBD)


# speedup vs baseline: 2.9540x; 1.0104x over previous
import functools

import jax
import jax.numpy as jnp
from jax import lax
from jax.experimental import pallas as pl
from jax.experimental.pallas import tpu as pltpu
from jax.experimental.pallas import tpu_sc as plsc

MAX_POS = 8192
N_EMBD = 1024
BATCH = 4
SEQ = 8192

_INFO = plsc.get_sparse_core_info()
_NC = _INFO.num_cores
_NS = _INFO.num_subcores
_NW = _NC * _NS
_B = BATCH * SEQ
_C = 8
_NBUF = 8


def _sc_gather(idx_flat, table, n_rows):
    per_w = n_rows // _NW
    n_chunks = per_w // _C
    mesh = plsc.VectorSubcoreMesh(core_axis_name="c", subcore_axis_name="s")

    @functools.partial(
        pl.kernel,
        mesh=mesh,
        out_type=jax.ShapeDtypeStruct((n_rows, N_EMBD), jnp.float32),
        scratch_types=[
            pltpu.VMEM((per_w,), jnp.int32),
            pltpu.VMEM((_NBUF, _C, N_EMBD), jnp.float32),
            pltpu.SemaphoreType.DMA((_NBUF,)),
            pltpu.SemaphoreType.DMA((_NBUF,)),
        ],
    )
    def k(idx_hbm, table_hbm, out_hbm, idx_v, rows_v, gsem, wsem):
        wid = lax.axis_index("s") * _NC + lax.axis_index("c")
        base = wid * per_w
        pltpu.sync_copy(idx_hbm.at[pl.ds(base, per_w)], idx_v)

        def start_gather(i, b):
            pltpu.async_copy(
                table_hbm.at[idx_v.at[pl.ds(i * _C, _C)]],
                rows_v.at[b],
                gsem.at[b],
            )

        def wait_gather(i, b):
            pltpu.make_async_copy(
                table_hbm.at[idx_v.at[pl.ds(i * _C, _C)]],
                rows_v.at[b],
                gsem.at[b],
            ).wait()

        def start_write(i, b):
            pltpu.async_copy(
                rows_v.at[b],
                out_hbm.at[pl.ds(base + i * _C, _C)],
                wsem.at[b],
            )

        def wait_write(i, b):
            pltpu.make_async_copy(
                rows_v.at[b],
                out_hbm.at[pl.ds(base + i * _C, _C)],
                wsem.at[b],
            ).wait()

        for b in range(_NBUF):
            start_gather(b, b)

        def body(g, carry):
            for b in range(_NBUF):
                i = g * _NBUF + b
                wait_gather(i, b)
                start_write(i, b)
                wait_write(i, b)
                start_gather(i + _NBUF, b)
            return carry

        lax.fori_loop(0, (n_chunks - _NBUF) // _NBUF, body, 0)

        tail = n_chunks - _NBUF
        for b in range(_NBUF):
            wait_gather(tail + b, b)
            start_write(tail + b, b)
        for b in range(_NBUF):
            wait_write(tail + b, b)

    return k(idx_flat, table)


def kernel(position_ids, table):
    idx_flat = position_ids.reshape(_B).astype(jnp.int32)
    out = _sc_gather(idx_flat, table, _B)
    return out.reshape(BATCH, SEQ, N_EMBD)
